# Initial kernel scaffold; baseline (speedup 1.0000x reference)
import jax
import jax.numpy as jnp
from jax.experimental import pallas as pl


def kernel(heatmap, feature_map, phase_embed, W1, b1, W2, b2, logit_scale, phase_embeds, next_targets, pos_mask, neg_mask, hard_neg_mask):
    raise NotImplementedError("write your pallas kernel here")


# speedup vs baseline: 6.2603x; 6.2603x over previous
import math

import jax
import jax.numpy as jnp
from jax.experimental import pallas as pl
from jax.experimental.pallas import tpu as pltpu

TOPK = 3
NMS_K = 5
SIGMA = 3.0
TEMP = 0.07
HARD_W = 2.0
NEG_MARGIN = 0.1
NEG_INF = -1e9

H = W = 512
V = 5
C = 128
D = 64
N = 2048
ROW_BLK = 256


def _peaks_kernel(hm_ref, out_ref):
    h = hm_ref[0]
    neg = jnp.float32(-jnp.inf)

    m = h
    for d in (1, 2):
        pad = jnp.full((d, W), neg, jnp.float32)
        up = jnp.concatenate([h[d:, :], pad], axis=0)
        dn = jnp.concatenate([pad, h[:-d, :]], axis=0)
        m = jnp.maximum(m, jnp.maximum(up, dn))
    p = m
    for d in (1, 2):
        pad = jnp.full((H, d), neg, jnp.float32)
        lf = jnp.concatenate([m[:, d:], pad], axis=1)
        rt = jnp.concatenate([pad, m[:, :-d]], axis=1)
        p = jnp.maximum(p, jnp.maximum(lf, rt))

    nms = jnp.where(h == p, h, 0.0)
    ids = (jax.lax.broadcasted_iota(jnp.int32, (H, W), 0) * W
           + jax.lax.broadcasted_iota(jnp.int32, (H, W), 1))
    big = jnp.int32(2 ** 30)

    found = []
    for _ in range(TOPK):
        mval = jnp.max(nms)
        idx = jnp.min(jnp.where(nms == mval, ids, big))
        found.append(idx)
        nms = jnp.where(ids == idx, neg, nms)

    li = jax.lax.broadcasted_iota(jnp.int32, (8, 128), 1)
    arr = jnp.where(li == 0, found[0], jnp.where(li == 1, found[1], found[2]))
    out_ref[0] = arr


def _run_peaks(hm0):
    return pl.pallas_call(
        _peaks_kernel,
        grid=(V,),
        in_specs=[pl.BlockSpec((1, H, W), lambda i: (i, 0, 0))],
        out_specs=pl.BlockSpec((1, 8, 128), lambda i: (i, 0, 0)),
        out_shape=jax.ShapeDtypeStruct((V, 8, 128), jnp.int32),
        compiler_params=pltpu.CompilerParams(
            dimension_semantics=("parallel",)),
    )(hm0)


def _select_kernel(yx_ref, fm_ref, w1_ref, b1_ref, w2_ref, b2_ref, peb_ref,
                   ls_ref, logit_ref, wt_ref, acc_ref):
    i = pl.program_id(0)
    npts = V * TOPK

    @pl.when(i == 0)
    def _():
        acc_ref[...] = jnp.zeros_like(acc_ref)

    y = yx_ref[i]
    x = yx_ref[npts + i]
    sub = jax.lax.rem(y, 8)
    lane = jax.lax.rem(x, 128)

    blk = fm_ref[0]
    sio = jax.lax.broadcasted_iota(jnp.int32, (8, 128), 0)
    lio = jax.lax.broadcasted_iota(jnp.int32, (8, 128), 1)
    msk = jnp.where((sio == sub) & (lio == lane), 1.0, 0.0)
    picked = jnp.sum(jnp.sum(blk * msk[None, :, :], axis=1), axis=1)
    acc_ref[pl.ds(i, 1)] = picked.reshape(1, 1, C)

    @pl.when(i == npts - 1)
    def _():
        s = acc_ref[...].reshape(16, C)
        h1 = jnp.maximum(
            jnp.dot(s, w1_ref[...], preferred_element_type=jnp.float32)
            + b1_ref[...], 0.0)
        pe = jnp.dot(h1, w2_ref[...],
                     preferred_element_type=jnp.float32) + b2_ref[...]
        nrm = jnp.sqrt(jnp.sum(pe * pe, axis=-1, keepdims=True))
        pe = pe / jnp.maximum(nrm, 1e-12)
        scale = jnp.minimum(jnp.exp(ls_ref[...]), 100.0)
        lg = jnp.dot(pe, peb_ref[...],
                     preferred_element_type=jnp.float32) * scale
        logit_ref[...] = lg

        rows = []
        for g in range(V):
            r0 = lg[3 * g:3 * g + 1, :]
            r1 = lg[3 * g + 1:3 * g + 2, :]
            r2 = lg[3 * g + 2:3 * g + 3, :]
            mx = jnp.maximum(r0, jnp.maximum(r1, r2))
            e0 = jnp.exp(r0 - mx)
            e1 = jnp.exp(r1 - mx)
            e2 = jnp.exp(r2 - mx)
            tot = e0 + e1 + e2
            rows += [e0 / tot, e1 / tot, e2 / tot]
        rows.append(jnp.zeros((1, 128), jnp.float32))
        wt_ref[...] = jnp.concatenate(rows, axis=0)


def _run_select(yx, feature_map, W1, b1, W2, b2, peb, ls):
    npts = V * TOPK
    grid_spec = pltpu.PrefetchScalarGridSpec(
        num_scalar_prefetch=1,
        grid=(npts,),
        in_specs=[
            pl.BlockSpec((1, C, 8, 128),
                         lambda i, yx: (i // TOPK, 0, yx[i] // 8,
                                        yx[npts + i] // 128)),
            pl.BlockSpec((C, D), lambda i, yx: (0, 0)),
            pl.BlockSpec((1, D), lambda i, yx: (0, 0)),
            pl.BlockSpec((D, D), lambda i, yx: (0, 0)),
            pl.BlockSpec((1, D), lambda i, yx: (0, 0)),
            pl.BlockSpec((D, 128), lambda i, yx: (0, 0)),
            pl.BlockSpec((1, 1), lambda i, yx: (0, 0)),
        ],
        out_specs=[
            pl.BlockSpec((16, 128), lambda i, yx: (0, 0)),
            pl.BlockSpec((16, 128), lambda i, yx: (0, 0)),
        ],
        scratch_shapes=[pltpu.VMEM((16, 1, C), jnp.float32)],
    )
    return pl.pallas_call(
        _select_kernel,
        grid_spec=grid_spec,
        out_shape=[jax.ShapeDtypeStruct((16, 128), jnp.float32),
                   jax.ShapeDtypeStruct((16, 128), jnp.float32)],
        compiler_params=pltpu.CompilerParams(
            dimension_semantics=("arbitrary",)),
    )(yx, feature_map, W1, b1, W2, b2, peb, ls)


def _reweight_kernel(yx_ref, w_ref, hm_ref, out_ref):
    i = pl.program_id(0)
    c = jax.lax.rem(i, V)
    npts = V * TOPK
    inv2s2 = 1.0 / (2.0 * SIGMA * SIGMA)

    lane = jax.lax.broadcasted_iota(jnp.int32, (1, H), 1).astype(jnp.float32)
    rrows, crows = [], []
    for k in range(TOPK):
        j = c * TOPK + k
        yf = yx_ref[j].astype(jnp.float32)
        xf = yx_ref[npts + j].astype(jnp.float32)
        wk = w_ref[j]
        dr = lane - yf
        dc = lane - xf
        rrows.append(wk * jnp.exp(-(dr * dr) * inv2s2))
        crows.append(jnp.exp(-(dc * dc) * inv2s2))
    z = jnp.zeros((8 - TOPK, H), jnp.float32)
    r8 = jnp.concatenate(rrows + [z], axis=0)
    c8 = jnp.concatenate(crows + [z], axis=0)
    bias = jax.lax.dot_general(r8, c8, (((0,), (0,)), ((), ())),
                               preferred_element_type=jnp.float32)
    out_ref[0] = hm_ref[0] + bias


def _run_reweight(hm_flat, yx, wts):
    grid_spec = pltpu.PrefetchScalarGridSpec(
        num_scalar_prefetch=2,
        grid=(hm_flat.shape[0],),
        in_specs=[pl.BlockSpec((1, H, W), lambda i, yx, w: (i, 0, 0))],
        out_specs=pl.BlockSpec((1, H, W), lambda i, yx, w: (i, 0, 0)),
    )
    return pl.pallas_call(
        _reweight_kernel,
        grid_spec=grid_spec,
        out_shape=jax.ShapeDtypeStruct(hm_flat.shape, jnp.float32),
        compiler_params=pltpu.CompilerParams(
            dimension_semantics=("parallel",)),
    )(yx, wts, hm_flat)


def _loss_kernel(pblk_ref, pfull_ref, pos_ref, neg_ref, hard_ref, nt_ref,
                 closs_ref, kloss_ref, acc_ref):
    i = pl.program_id(0)
    nblk = N // ROW_BLK
    ln_hard = math.log(HARD_W)

    @pl.when(i == 0)
    def _():
        acc_ref[0] = 0.0
        acc_ref[1] = 0.0
        p = pfull_ref[...]
        a = p[:-1, :]
        b = p[1:, :]
        dots = jnp.sum(a * b, axis=-1, keepdims=True)
        na = jnp.sqrt(jnp.sum(a * a, axis=-1, keepdims=True))
        nb = jnp.sqrt(jnp.sum(b * b, axis=-1, keepdims=True))
        cos = dots / jnp.maximum(na * nb, 1e-8)
        nt = nt_ref[...]
        dt = nt[:-1, :] - nt[1:, :]
        td = jnp.sqrt(jnp.sum(dt * dt, axis=-1, keepdims=True))
        same = jnp.where(td < NEG_MARGIN, 1.0, 0.0)
        val = same * jnp.maximum(same * 0.9 - cos, 0.0)
        kloss_ref[...] = (jnp.sum(val) / (N - 1.0)).reshape(1, 1)

    sim = jax.lax.dot_general(pblk_ref[...], pfull_ref[...],
                              (((1,), (1,)), ((), ())),
                              preferred_element_type=jnp.float32) / TEMP
    pos = pos_ref[...]
    neg = neg_ref[...]
    hard = hard_ref[...]

    pos_sim = jnp.where(pos, sim, NEG_INF)
    mp = jnp.max(pos_sim, axis=-1, keepdims=True)
    lse_p = mp + jnp.log(jnp.sum(jnp.exp(pos_sim - mp), axis=-1,
                                 keepdims=True))
    neg_sim = jnp.where(neg, sim, NEG_INF) + jnp.where(hard, ln_hard, 0.0)
    mn = jnp.max(neg_sim, axis=-1, keepdims=True)
    lse_n = mn + jnp.log(jnp.sum(jnp.exp(neg_sim - mn), axis=-1,
                                 keepdims=True))
    hi = jnp.maximum(lse_p, lse_n)
    lo = jnp.minimum(lse_p, lse_n)
    log_denom = hi + jnp.log1p(jnp.exp(lo - hi))

    row_any = jnp.max(jnp.where(pos, 1.0, 0.0), axis=-1, keepdims=True)
    per = -(lse_p - log_denom)
    acc_ref[0] = acc_ref[0] + jnp.sum(jnp.where(row_any > 0.0, per, 0.0))
    acc_ref[1] = acc_ref[1] + jnp.sum(row_any)

    @pl.when(i == nblk - 1)
    def _():
        closs_ref[...] = (acc_ref[0] / jnp.maximum(acc_ref[1], 1.0)).reshape(1, 1)


def _run_losses(pe_pad, pos_mask, neg_mask, hard_neg_mask, nt_pad):
    nblk = N // ROW_BLK
    return pl.pallas_call(
        _loss_kernel,
        grid=(nblk,),
        in_specs=[
            pl.BlockSpec((ROW_BLK, 128), lambda i: (i, 0)),
            pl.BlockSpec((N, 128), lambda i: (0, 0)),
            pl.BlockSpec((ROW_BLK, N), lambda i: (i, 0)),
            pl.BlockSpec((ROW_BLK, N), lambda i: (i, 0)),
            pl.BlockSpec((ROW_BLK, N), lambda i: (i, 0)),
            pl.BlockSpec((N, 128), lambda i: (0, 0)),
        ],
        out_specs=[
            pl.BlockSpec((1, 1), lambda i: (0, 0)),
            pl.BlockSpec((1, 1), lambda i: (0, 0)),
        ],
        out_shape=[jax.ShapeDtypeStruct((1, 1), jnp.float32),
                   jax.ShapeDtypeStruct((1, 1), jnp.float32)],
        scratch_shapes=[pltpu.SMEM((2,), jnp.float32)],
        compiler_params=pltpu.CompilerParams(
            dimension_semantics=("arbitrary",)),
    )(pe_pad, pe_pad, pos_mask, neg_mask, hard_neg_mask, nt_pad)


def kernel(heatmap, feature_map, phase_embed, W1, b1, W2, b2, logit_scale,
           phase_embeds, next_targets, pos_mask, neg_mask, hard_neg_mask):
    bs, nc = heatmap.shape[0], heatmap.shape[1]

    pk = _run_peaks(heatmap[0])
    idx = pk[:, 0, :TOPK]
    rows = (idx // W).reshape(-1)
    cols = (idx % W).reshape(-1)
    yx = jnp.concatenate([rows, cols])

    peb = jnp.broadcast_to(phase_embed.reshape(D, 1), (D, 128))
    lg, wt = _run_select(yx, feature_map, W1, b1.reshape(1, D), W2,
                         b2.reshape(1, D), peb, logit_scale.reshape(1, 1))
    peak_logits = lg[:V * TOPK, 0].reshape(V, TOPK)
    wts = wt[:V * TOPK, 0]

    hm_flat = heatmap.reshape(bs * nc, H, W)
    rw = _run_reweight(hm_flat, yx, wts)
    reweighted = rw.reshape(bs, nc, 1, H, W)

    nt_pad = jnp.pad(next_targets, ((0, 0), (0, 128 - next_targets.shape[1])))
    pe_pad = jnp.pad(phase_embeds, ((0, 0), (0, 128 - D)))
    cl, kl = _run_losses(pe_pad, pos_mask, neg_mask, hard_neg_mask, nt_pad)

    return reweighted, peak_logits, cl[0, 0], kl[0, 0]
